# baseline (speedup 1.0000x reference)
import functools

import jax
import jax.numpy as jnp
from jax import lax
from jax.experimental import pallas as pl
from jax.experimental.pallas import tpu as pltpu
from jax.experimental.pallas import tpu_sc as plsc

L_SEQ = 4096
D_EMB = 1024
_SPLIT = 3072
_NC, _NS = 2, 16
_NW = _NC * _NS
_SC_ROWS = L_SEQ - _SPLIT
_ROWS_PER_W = _SC_ROWS // _NW

_mesh = plsc.VectorSubcoreMesh(core_axis_name="c", subcore_axis_name="s")


@functools.partial(
    pl.kernel,
    out_type=jax.ShapeDtypeStruct((_SC_ROWS, D_EMB), jnp.float32),
    mesh=_mesh,
    scratch_types=[
        pltpu.VMEM((_ROWS_PER_W, D_EMB), jnp.float32),
        pltpu.SemaphoreType.DMA,
        pltpu.SemaphoreType.DMA,
    ],
)
def _sc_copy_tail(table_hbm, out_hbm, buf, lsem, ssem):
    wid = lax.axis_index("s") * _NC + lax.axis_index("c")
    base = wid * _ROWS_PER_W
    pltpu.make_async_copy(
        table_hbm.at[pl.ds(_SPLIT + base, _ROWS_PER_W)], buf, lsem
    ).start()
    pltpu.make_async_copy(
        table_hbm.at[pl.ds(_SPLIT + base, _ROWS_PER_W)], buf, lsem
    ).wait()
    pltpu.make_async_copy(buf, out_hbm.at[pl.ds(base, _ROWS_PER_W)], ssem).start()
    pltpu.make_async_copy(buf, out_hbm.at[pl.ds(base, _ROWS_PER_W)], ssem).wait()


def _tc_copy_block(emb_ref, out_ref):
    out_ref[...] = emb_ref[...]


def kernel(input_ids, embedding):
    del input_ids
    sc_part = _sc_copy_tail(embedding)
    tc_part = pl.pallas_call(
        _tc_copy_block,
        grid=(_SPLIT // 512,),
        in_specs=[pl.BlockSpec((512, D_EMB), lambda i: (i, 0))],
        out_specs=pl.BlockSpec((512, D_EMB), lambda i: (i, 0)),
        out_shape=jax.ShapeDtypeStruct((_SPLIT, D_EMB), embedding.dtype),
    )(embedding)
    return jnp.concatenate([tc_part, sc_part], axis=0)[None]


# speedup vs baseline: 1.3285x; 1.3285x over previous
import functools

import jax
import jax.numpy as jnp
from jax import lax
from jax.experimental import pallas as pl
from jax.experimental.pallas import tpu as pltpu
from jax.experimental.pallas import tpu_sc as plsc

L_SEQ = 4096
D_EMB = 1024
_NC, _NS = 2, 16
_NW = _NC * _NS
_ROWS_PER_W = L_SEQ // _NW

_CHUNK = 16
_NBUF = 7
_NCHUNK = _ROWS_PER_W // _CHUNK

_mesh = plsc.VectorSubcoreMesh(core_axis_name="c", subcore_axis_name="s")


@functools.partial(
    pl.kernel,
    out_type=jax.ShapeDtypeStruct((L_SEQ, D_EMB), jnp.float32),
    mesh=_mesh,
    scratch_types=(
        [pltpu.VMEM((_CHUNK, D_EMB), jnp.float32)] * _NBUF
        + [pltpu.SemaphoreType.DMA] * (2 * _NBUF)
    ),
)
def _sc_copy(table_hbm, out_hbm, *scratch):
    bufs = scratch[:_NBUF]
    lsems = scratch[_NBUF:2 * _NBUF]
    ssems = scratch[2 * _NBUF:]
    wid = lax.axis_index("s") * _NC + lax.axis_index("c")
    base = wid * _ROWS_PER_W

    def load(i):
        return pltpu.make_async_copy(
            table_hbm.at[pl.ds(base + i * _CHUNK, _CHUNK)],
            bufs[i % _NBUF], lsems[i % _NBUF])

    def store(i):
        return pltpu.make_async_copy(
            bufs[i % _NBUF],
            out_hbm.at[pl.ds(base + i * _CHUNK, _CHUNK)], ssems[i % _NBUF])

    for i in range(_NBUF):
        load(i).start()
    for i in range(_NCHUNK):
        load(i).wait()
        store(i).start()
        if i + _NBUF < _NCHUNK:
            store(i).wait()
            load(i + _NBUF).start()
    for i in range(_NCHUNK - _NBUF, _NCHUNK):
        store(i).wait()


def kernel(input_ids, embedding):
    del input_ids
    return _sc_copy(embedding)[None]
